# Initial kernel scaffold; baseline (speedup 1.0000x reference)
import jax
import jax.numpy as jnp
from jax.experimental import pallas as pl


def kernel(input, embedding_weight):
    raise NotImplementedError("write your pallas kernel here")


# speedup vs baseline: 17.7057x; 17.7057x over previous
import jax
import jax.numpy as jnp
from jax import lax
from jax.experimental import pallas as pl
from jax.experimental.pallas import tpu as pltpu
from jax.experimental.pallas import tpu_sc as plsc

B, L, D, V = 16384, 200, 128, 100000
NC, NS = 2, 16
NW = NC * NS
RPW = B // NW
G = 32
NG = RPW // G
C0, C1 = 104, 96
DV = D // 16


def _body(idx_hbm, tab_hbm, out_hbm, idx_v, rows_v, out_v, sem0, sem1):
    cid = lax.axis_index("c")
    sid = lax.axis_index("s")
    wid = sid * NC + cid
    base = wid * RPW

    sems = (sem0, sem1)

    def chunk_copies(rl, slot):
        off0 = pl.multiple_of(rl * L, 8)
        off1 = pl.multiple_of(rl * L + C0, 8)
        c0 = pltpu.make_async_copy(
            tab_hbm.at[idx_v.at[pl.ds(off0, C0)]],
            rows_v.at[slot, pl.ds(0, C0)], sems[slot])
        c1 = pltpu.make_async_copy(
            tab_hbm.at[idx_v.at[pl.ds(off1, C1)]],
            rows_v.at[slot, pl.ds(C0, C1)], sems[slot])
        return c0, c1

    def start_row(rl, slot):
        for c in chunk_copies(rl, slot):
            c.start()

    def wait_row(rl, slot):
        for c in chunk_copies(rl, slot):
            c.wait()

    def compute_row(rl, slot):
        def red(l, accs):
            return tuple(
                jnp.maximum(a, rows_v[slot, l, pl.ds(16 * d, 16)])
                for d, a in enumerate(accs))
        accs = tuple(rows_v[slot, 0, pl.ds(16 * d, 16)] for d in range(DV))
        accs = lax.fori_loop(1, L, red, accs, unroll=4)
        for d in range(DV):
            out_v[rl, pl.ds(16 * d, 16)] = accs[d]

    def group(g, carry):
        row0 = pl.multiple_of((base + g * G) * L, 8)
        pltpu.sync_copy(idx_hbm.at[pl.ds(row0, G * L)], idx_v)
        start_row(0, 0)

        def pair(p, c):
            r0 = 2 * p
            r1 = r0 + 1
            start_row(r1, 1)
            wait_row(r0, 0)
            compute_row(r0, 0)

            @pl.when(r1 + 1 < G)
            def _():
                start_row(r1 + 1, 0)

            wait_row(r1, 1)
            compute_row(r1, 1)
            return c

        lax.fori_loop(0, G // 2, pair, 0)
        out0 = pl.multiple_of(base + g * G, 8)
        pltpu.sync_copy(out_v, out_hbm.at[pl.ds(out0, G)])
        return carry

    lax.fori_loop(0, NG, group, 0)


def kernel(input, embedding_weight):
    idx = jnp.asarray(input, jnp.int32).reshape(-1)
    mesh = plsc.VectorSubcoreMesh(
        core_axis_name="c", subcore_axis_name="s",
        num_cores=NC, num_subcores=NS)
    f = pl.kernel(
        _body,
        out_type=jax.ShapeDtypeStruct((B, D), jnp.float32),
        mesh=mesh,
        scratch_types=[
            pltpu.VMEM((G * L,), jnp.int32),
            pltpu.VMEM((2, L, D), jnp.float32),
            pltpu.VMEM((G, D), jnp.float32),
            pltpu.SemaphoreType.DMA,
            pltpu.SemaphoreType.DMA,
        ],
    )
    return f(idx, embedding_weight)
